# speedup vs baseline: 7.6967x; 7.6967x over previous
import functools

import jax
import jax.numpy as jnp
from jax.experimental import pallas as pl
from jax.experimental.pallas import tpu as pltpu

BATCH = 4096
N_IN = 2048
H1 = 1024
H2 = 512
K_SPARSE = int(H2 * 0.05)

BLOCK_B = 512


def _fused_kernel(x_ref, we1_ref, be1_ref, we2_ref, be2_ref,
                  wd0_ref, bd0_ref, wd1_ref, bd1_ref, out_ref):
    x = x_ref[...]
    h1 = jax.lax.dot_general(
        x, we1_ref[...], (((1,), (1,)), ((), ())),
        preferred_element_type=jnp.float32)
    h1 = jax.nn.sigmoid(h1 + be1_ref[...])
    h2 = jax.lax.dot_general(
        h1, we2_ref[...], (((1,), (1,)), ((), ())),
        preferred_element_type=jnp.float32)
    h2 = jax.nn.sigmoid(h2 + be2_ref[...])

    neg_inf = jnp.float32(-jnp.inf)

    def body(_, t):
        m = jnp.where(h2 < t, h2, neg_inf)
        return jnp.max(m, axis=1, keepdims=True)

    t0 = jnp.full((h2.shape[0], 1), jnp.inf, dtype=jnp.float32)
    t = jax.lax.fori_loop(0, K_SPARSE, body, t0)
    h2m = jnp.where(h2 >= t, h2, 0.0)

    d1 = jax.lax.dot_general(
        h2m, wd1_ref[...], (((1,), (1,)), ((), ())),
        preferred_element_type=jnp.float32)
    d1 = jax.nn.sigmoid(d1 + bd1_ref[...])
    out = jax.lax.dot_general(
        d1, wd0_ref[...], (((1,), (1,)), ((), ())),
        preferred_element_type=jnp.float32)
    out_ref[...] = out + bd0_ref[...]


@jax.jit
def kernel(input, We1, be1, We2, be2, Wd0, bd0, Wd1, bd1):
    b1 = be1.reshape(1, H1)
    b2 = be2.reshape(1, H2)
    b0 = bd0.reshape(1, N_IN)
    bd1r = bd1.reshape(1, H1)
    grid = (BATCH // BLOCK_B,)
    const = lambda i: (0, 0)
    return pl.pallas_call(
        _fused_kernel,
        grid=grid,
        in_specs=[
            pl.BlockSpec((BLOCK_B, N_IN), lambda i: (i, 0)),
            pl.BlockSpec((H1, N_IN), const),
            pl.BlockSpec((1, H1), const),
            pl.BlockSpec((H2, H1), const),
            pl.BlockSpec((1, H2), const),
            pl.BlockSpec((N_IN, H1), const),
            pl.BlockSpec((1, N_IN), const),
            pl.BlockSpec((H1, H2), const),
            pl.BlockSpec((1, H1), const),
        ],
        out_specs=pl.BlockSpec((BLOCK_B, N_IN), lambda i: (i, 0)),
        out_shape=jax.ShapeDtypeStruct((BATCH, N_IN), jnp.float32),
    )(input, We1, b1, We2, b2, Wd0, b0, Wd1, bd1r)
